# Initial kernel scaffold; baseline (speedup 1.0000x reference)
import jax
import jax.numpy as jnp
from jax.experimental import pallas as pl


def kernel(graph, Xv, W1, W2, W3, W4, W5, W6, W7):
    raise NotImplementedError("write your pallas kernel here")


# speedup vs baseline: 1.0344x; 1.0344x over previous
import jax
import jax.numpy as jnp
from jax.experimental import pallas as pl
from jax.experimental.pallas import tpu as pltpu

EMB = 32
N = 4096
BLK = 256
NBLK = N // BLK


def _mmT(x, w):
    return jax.lax.dot_general(x, w, (((1,), (1,)), ((), ())),
                               preferred_element_type=jnp.float32)


def _body(graph_ref, xv_ref, w1t_ref, w2_ref, w3_ref, w4t_ref, w5a_ref,
          w5b_ref, w6_ref, w7_ref, out_ref, emb_a, emb_b, c_ref, r6w_ref):
    p = pl.program_id(0)
    i = pl.program_id(1)
    row = pl.ds(i * BLK, BLK)

    @pl.when(p == 0)
    def _init():
        g = graph_ref[...]
        r = jnp.sum(g, axis=1, keepdims=True)
        a = xv_ref[row, :] * w1t_ref[...]
        ut = _mmT(w4t_ref[...], w3_ref[...])
        cb = a + r * ut
        c_ref[row, :] = cb
        emb_a[row, :] = jnp.maximum(cb, 0.0)

    def _step(src, dst):
        g = graph_ref[...]
        ns = jnp.dot(g, src[...], preferred_element_type=jnp.float32)
        v2 = _mmT(ns, w2_ref[...])
        dst[row, :] = jnp.maximum(c_ref[row, :] + v2, 0.0)

    pl.when(p == 1)(lambda: _step(emb_a, emb_b))
    pl.when(p == 2)(lambda: _step(emb_b, emb_a))
    pl.when(p == 3)(lambda: _step(emb_a, emb_b))

    @pl.when((p == 4) & (i == 0))
    def _glob():
        es = jnp.sum(emb_b[...], axis=0, keepdims=True)
        r6 = jnp.maximum(_mmT(es, w6_ref[...]), 0.0)
        r6w_ref[...] = r6 * w5a_ref[...]

    @pl.when(p == 4)
    def _out():
        r7 = jnp.maximum(_mmT(emb_b[row, :], w7_ref[...]), 0.0)
        out_ref[...] = jnp.sum(r7 * w5b_ref[...] + r6w_ref[...],
                               axis=1, keepdims=True)


def kernel(graph, Xv, W1, W2, W3, W4, W5, W6, W7):
    g2 = graph.reshape(N, N)
    xv2 = Xv.reshape(N, 1)
    w1t = W1.reshape(1, EMB)
    w4t = W4.reshape(1, EMB)
    w5a = W5[:, :EMB]
    w5b = W5[:, EMB:]

    full = lambda shape: pl.BlockSpec(shape, lambda p, i: (0, 0))
    out = pl.pallas_call(
        _body,
        grid=(5, NBLK),
        in_specs=[
            pl.BlockSpec((BLK, N), lambda p, i: (jnp.where(p >= 4, 0, i), 0)),
            full((N, 1)),
            full((1, EMB)),
            full((EMB, EMB)),
            full((EMB, EMB)),
            full((1, EMB)),
            full((1, EMB)),
            full((1, EMB)),
            full((EMB, EMB)),
            full((EMB, EMB)),
        ],
        out_specs=pl.BlockSpec((BLK, 1), lambda p, i: (jnp.where(p == 4, i, 0), 0)),
        out_shape=jax.ShapeDtypeStruct((N, 1), jnp.float32),
        scratch_shapes=[
            pltpu.VMEM((N, EMB), jnp.float32),
            pltpu.VMEM((N, EMB), jnp.float32),
            pltpu.VMEM((N, EMB), jnp.float32),
            pltpu.VMEM((1, EMB), jnp.float32),
        ],
        compiler_params=pltpu.CompilerParams(
            dimension_semantics=("arbitrary", "arbitrary")),
    )(g2, xv2, w1t, W2, W3, w4t, w5a, w5b, W6, W7)
    return out.reshape(1, N)


# speedup vs baseline: 1.3420x; 1.2973x over previous
import jax
import jax.numpy as jnp
from jax.experimental import pallas as pl
from jax.experimental.pallas import tpu as pltpu

EMB = 32
N = 4096
BLK = 256
NBLK = N // BLK


def _mmT(x, w):
    return jax.lax.dot_general(x, w, (((1,), (1,)), ((), ())),
                               preferred_element_type=jnp.float32)


def _body(graph_ref, xv_ref, w1t_ref, w2_ref, w3_ref, w4t_ref, w5a_ref,
          w5b_ref, w6_ref, w7_ref, out_ref, gb_ref, emb_a, emb_b, c_ref,
          r6w_ref):
    p = pl.program_id(0)
    i = pl.program_id(1)
    row = pl.ds(i * BLK, BLK)

    @pl.when(p == 0)
    def _init():
        g = graph_ref[...]
        gb_ref[row, :] = g.astype(jnp.bfloat16)
        r = jnp.sum(g, axis=1, keepdims=True)
        a = xv_ref[row, :] * w1t_ref[...]
        ut = _mmT(w4t_ref[...], w3_ref[...])
        cb = a + r * ut
        c_ref[row, :] = cb
        emb_a[row, :] = jnp.maximum(cb, 0.0)

    def _step(src, dst):
        gb = gb_ref[row, :]
        ns = jnp.dot(gb, src[...].astype(jnp.bfloat16),
                     preferred_element_type=jnp.float32)
        v2 = _mmT(ns, w2_ref[...])
        dst[row, :] = jnp.maximum(c_ref[row, :] + v2, 0.0)

    pl.when(p == 1)(lambda: _step(emb_a, emb_b))
    pl.when(p == 2)(lambda: _step(emb_b, emb_a))
    pl.when(p == 3)(lambda: _step(emb_a, emb_b))

    @pl.when((p == 4) & (i == 0))
    def _glob():
        es = jnp.sum(emb_b[...], axis=0, keepdims=True)
        r6 = jnp.maximum(_mmT(es, w6_ref[...]), 0.0)
        r6w_ref[...] = r6 * w5a_ref[...]

    @pl.when(p == 4)
    def _out():
        r7 = jnp.maximum(_mmT(emb_b[row, :], w7_ref[...]), 0.0)
        out_ref[...] = jnp.sum(r7 * w5b_ref[...] + r6w_ref[...],
                               axis=1, keepdims=True)


def kernel(graph, Xv, W1, W2, W3, W4, W5, W6, W7):
    g2 = graph.reshape(N, N)
    xv2 = Xv.reshape(N, 1)
    w1t = W1.reshape(1, EMB)
    w4t = W4.reshape(1, EMB)
    w5a = W5[:, :EMB]
    w5b = W5[:, EMB:]

    full = lambda shape: pl.BlockSpec(shape, lambda p, i: (0, 0))
    out = pl.pallas_call(
        _body,
        grid=(5, NBLK),
        in_specs=[
            pl.BlockSpec((BLK, N), lambda p, i: (jnp.where(p == 0, i, NBLK - 1), 0)),
            full((N, 1)),
            full((1, EMB)),
            full((EMB, EMB)),
            full((EMB, EMB)),
            full((1, EMB)),
            full((1, EMB)),
            full((1, EMB)),
            full((EMB, EMB)),
            full((EMB, EMB)),
        ],
        out_specs=pl.BlockSpec((BLK, 1), lambda p, i: (jnp.where(p == 4, i, 0), 0)),
        out_shape=jax.ShapeDtypeStruct((N, 1), jnp.float32),
        scratch_shapes=[
            pltpu.VMEM((N, N), jnp.bfloat16),
            pltpu.VMEM((N, EMB), jnp.float32),
            pltpu.VMEM((N, EMB), jnp.float32),
            pltpu.VMEM((N, EMB), jnp.float32),
            pltpu.VMEM((1, EMB), jnp.float32),
        ],
        compiler_params=pltpu.CompilerParams(
            dimension_semantics=("arbitrary", "arbitrary")),
    )(g2, xv2, w1t, W2, W3, w4t, w5a, w5b, W6, W7)
    return out.reshape(1, N)


# speedup vs baseline: 1.6470x; 1.2273x over previous
import jax
import jax.numpy as jnp
from jax.experimental import pallas as pl
from jax.experimental.pallas import tpu as pltpu

EMB = 32
N = 4096
BLK = 512
NBLK = N // BLK


def _mmT(x, w):
    return jax.lax.dot_general(x, w, (((1,), (1,)), ((), ())),
                               preferred_element_type=jnp.float32)


def _body(graph_ref, xv_ref, w1t_ref, w2_ref, w3_ref, w4t_ref, w5a_ref,
          w5b_ref, w6_ref, w7_ref, out_ref, gb_ref, ebf_a, ebf_b, emb_f,
          c_ref, r6w_ref):
    p = pl.program_id(0)
    i = pl.program_id(1)
    row = pl.ds(i * BLK, BLK)

    @pl.when(p == 0)
    def _init():
        g = graph_ref[...]
        gb_ref[row, :] = g.astype(jnp.bfloat16)
        r = jnp.sum(g, axis=1, keepdims=True)
        a = xv_ref[row, :] * w1t_ref[...]
        ut = _mmT(w4t_ref[...], w3_ref[...])
        cb = a + r * ut
        c_ref[row, :] = cb
        ebf_a[row, :] = jnp.maximum(cb, 0.0).astype(jnp.bfloat16)

    def _step(src, dst, last):
        gb = gb_ref[row, :]
        ns = jnp.dot(gb, src[...], preferred_element_type=jnp.float32)
        v2 = _mmT(ns, w2_ref[...])
        e = jnp.maximum(c_ref[row, :] + v2, 0.0)
        dst[row, :] = e.astype(jnp.bfloat16)
        if last:
            emb_f[row, :] = e

    pl.when(p == 1)(lambda: _step(ebf_a, ebf_b, False))
    pl.when(p == 2)(lambda: _step(ebf_b, ebf_a, False))
    pl.when(p == 3)(lambda: _step(ebf_a, ebf_b, True))

    @pl.when((p == 4) & (i == 0))
    def _glob():
        es = jnp.sum(emb_f[...], axis=0, keepdims=True)
        r6 = jnp.maximum(_mmT(es, w6_ref[...]), 0.0)
        r6w_ref[...] = r6 * w5a_ref[...]

    @pl.when(p == 4)
    def _out():
        r7 = jnp.maximum(_mmT(emb_f[row, :], w7_ref[...]), 0.0)
        out_ref[...] = jnp.sum(r7 * w5b_ref[...] + r6w_ref[...],
                               axis=1, keepdims=True)


def kernel(graph, Xv, W1, W2, W3, W4, W5, W6, W7):
    g2 = graph.reshape(N, N)
    xv2 = Xv.reshape(N, 1)
    w1t = W1.reshape(1, EMB)
    w4t = W4.reshape(1, EMB)
    w5a = W5[:, :EMB]
    w5b = W5[:, EMB:]

    full = lambda shape: pl.BlockSpec(shape, lambda p, i: (0, 0))
    out = pl.pallas_call(
        _body,
        grid=(5, NBLK),
        in_specs=[
            pl.BlockSpec((BLK, N), lambda p, i: (jnp.where(p == 0, i, NBLK - 1), 0)),
            full((N, 1)),
            full((1, EMB)),
            full((EMB, EMB)),
            full((EMB, EMB)),
            full((1, EMB)),
            full((1, EMB)),
            full((1, EMB)),
            full((EMB, EMB)),
            full((EMB, EMB)),
        ],
        out_specs=pl.BlockSpec((BLK, 1), lambda p, i: (jnp.where(p == 4, i, 0), 0)),
        out_shape=jax.ShapeDtypeStruct((N, 1), jnp.float32),
        scratch_shapes=[
            pltpu.VMEM((N, N), jnp.bfloat16),
            pltpu.VMEM((N, EMB), jnp.bfloat16),
            pltpu.VMEM((N, EMB), jnp.bfloat16),
            pltpu.VMEM((N, EMB), jnp.float32),
            pltpu.VMEM((N, EMB), jnp.float32),
            pltpu.VMEM((1, EMB), jnp.float32),
        ],
        compiler_params=pltpu.CompilerParams(
            dimension_semantics=("arbitrary", "arbitrary")),
    )(g2, xv2, w1t, W2, W3, w4t, w5a, w5b, W6, W7)
    return out.reshape(1, N)
